# Initial kernel scaffold; baseline (speedup 1.0000x reference)
import jax
import jax.numpy as jnp
from jax.experimental import pallas as pl


def kernel(node_emb, W, W_r, a, rel_emb, edge_index, edge_type):
    raise NotImplementedError("write your pallas kernel here")


# speedup vs baseline: 3.0361x; 3.0361x over previous
import functools

import jax
import jax.numpy as jnp
from jax import lax
from jax.experimental import pallas as pl
from jax.experimental.pallas import tpu as pltpu
from jax.experimental.pallas import tpu_sc as plsc

_NU = 2000
_N = 10000
_DIM = 128
_NREL = 16
_E = 320000

_NC, _NS = 2, 16
_NW = _NC * _NS
_EPW = _E // _NW
_C = 80
_NCHUNK = _EPW // _C
_NPAD = 10240
_STRIPE = _NPAD // _NS


def _pre_body(ne_ref, w_ref, wr_ref, a_ref, rel_ref, xt_ref, sc_ref, rs_ref):
    xt = jnp.dot(ne_ref[...], w_ref[...], preferred_element_type=jnp.float32)
    xt_ref[...] = xt
    a = a_ref[...]
    a13 = jnp.concatenate([a[0:128], a[256:384]], axis=1)
    sc_ref[...] = jnp.dot(xt, a13, preferred_element_type=jnp.float32)
    rxt = jnp.dot(rel_ref[...], wr_ref[...], preferred_element_type=jnp.float32)
    rs_ref[...] = jnp.dot(rxt, a[128:256], preferred_element_type=jnp.float32)


_pre = pl.pallas_call(
    _pre_body,
    out_shape=[
        jax.ShapeDtypeStruct((_N, _DIM), jnp.float32),
        jax.ShapeDtypeStruct((_N, 2), jnp.float32),
        jax.ShapeDtypeStruct((_NREL, 1), jnp.float32),
    ],
)


def _sc_body(xt_hbm, s1_hbm, s3_hbm, rs_hbm, src_hbm, dst_hbm, et_hbm,
             msg_out, att_out,
             s1v, s3v, relv, src_c, dst_c, et_c, attb, rows_v,
             z2d, z1d, msg_sh, att_sh, gsem):
    cid = lax.axis_index("c")
    sid = lax.axis_index("s")
    wid = cid * _NS + sid

    zv = jnp.zeros((16,), jnp.float32)

    def _z2(i, carry):
        z2d[i // 8, pl.ds((i % 8) * 16, 16)] = zv
        return carry

    lax.fori_loop(0, 160 * 8, _z2, 0)

    def _z1(i, carry):
        z1d[pl.ds(i * 16, 16)] = zv
        return carry

    lax.fori_loop(0, _STRIPE // 16, _z1, 0)

    def _zc(i, carry):
        pltpu.sync_copy(z2d, msg_sh.at[pl.ds(sid * _STRIPE + i * 160, 160)])
        return carry

    lax.fori_loop(0, _STRIPE // 160, _zc, 0)
    pltpu.sync_copy(z1d, att_sh.at[pl.ds(sid * _STRIPE, _STRIPE)])
    plsc.subcore_barrier()

    def _chunk(c, carry):
        base = wid * _EPW + c * _C
        pltpu.sync_copy(src_hbm.at[pl.ds(base, _C)], src_c)
        pltpu.sync_copy(dst_hbm.at[pl.ds(base, _C)], dst_c)
        pltpu.sync_copy(et_hbm.at[pl.ds(base, _C)], et_c)

        g1 = pltpu.async_copy(s1_hbm.at[src_c], s1v, gsem)
        g2 = pltpu.async_copy(s3_hbm.at[dst_c], s3v, gsem)
        g3 = pltpu.async_copy(rs_hbm.at[et_c], relv, gsem)
        g4 = pltpu.async_copy(xt_hbm.at[src_c], rows_v, gsem)
        g1.wait()
        g2.wait()
        g3.wait()

        def _att(j, carry2):
            sl = pl.ds(j * 16, 16)
            v = s1v[sl] + s3v[sl] + relv[sl]
            v = jnp.where(v >= 0.0, v, 0.2 * v)
            attb[sl] = jnp.exp(v)
            return carry2

        lax.fori_loop(0, _C // 16, _att, 0)

        g4.wait()

        def _scale(j, carry2):
            a16 = attb[pl.ds(j * 16, 16)]
            for l in range(16):
                av = jnp.full((16,), a16[l], jnp.float32)
                k = j * 16 + l

                def _mul(q, carry3, k=k, av=av):
                    rows_v[k, pl.ds(q * 16, 16)] = rows_v[k, pl.ds(q * 16, 16)] * av
                    return carry3

                lax.fori_loop(0, _DIM // 16, _mul, 0)
            return carry2

        lax.fori_loop(0, _C // 16, _scale, 0)

        pltpu.sync_copy(rows_v, msg_sh.at[dst_c], add=True)
        pltpu.sync_copy(attb, att_sh.at[dst_c], add=True)
        return carry

    lax.fori_loop(0, _NCHUNK, _chunk, 0)

    plsc.subcore_barrier()
    pltpu.sync_copy(msg_sh.at[pl.ds(sid * _STRIPE, _STRIPE)],
                    msg_out.at[cid, pl.ds(sid * _STRIPE, _STRIPE)])
    pltpu.sync_copy(att_sh.at[pl.ds(sid * _STRIPE, _STRIPE)],
                    att_out.at[pl.ds(cid * _NPAD + sid * _STRIPE, _STRIPE)])


_sc = functools.partial(
    pl.kernel,
    out_type=[
        jax.ShapeDtypeStruct((_NC, _NPAD, _DIM), jnp.float32),
        jax.ShapeDtypeStruct((_NC * _NPAD,), jnp.float32),
    ],
    mesh=plsc.VectorSubcoreMesh(core_axis_name="c", subcore_axis_name="s"),
    scratch_types=[
        pltpu.VMEM((_C,), jnp.float32),
        pltpu.VMEM((_C,), jnp.float32),
        pltpu.VMEM((_C,), jnp.float32),
        pltpu.VMEM((_C,), jnp.int32),
        pltpu.VMEM((_C,), jnp.int32),
        pltpu.VMEM((_C,), jnp.int32),
        pltpu.VMEM((_C,), jnp.float32),
        pltpu.VMEM((_C, _DIM), jnp.float32),
        pltpu.VMEM((160, _DIM), jnp.float32),
        pltpu.VMEM((_STRIPE,), jnp.float32),
        pltpu.VMEM_SHARED((_NPAD, _DIM), jnp.float32),
        pltpu.VMEM_SHARED((_NPAD,), jnp.float32),
        pltpu.SemaphoreType.DMA,
    ],
)(_sc_body)


def _post_body(msg_ref, att_ref, ne_ref, out_ref):
    msg = msg_ref[0, :_N, :] + msg_ref[1, :_N, :]
    att = att_ref[:_NPAD] + att_ref[_NPAD:]
    out = msg / (att[:_N][:, None] + 1e-10)
    x1 = jnp.where(out > 0.0, out, jnp.exp(out) - 1.0)
    nrm = jnp.sqrt(jnp.sum(x1 * x1, axis=-1, keepdims=True))
    x1 = x1 / jnp.maximum(nrm, 1e-12)
    out_ref[...] = (ne_ref[...] + x1) * 0.5


_post = pl.pallas_call(
    _post_body,
    out_shape=jax.ShapeDtypeStruct((_N, _DIM), jnp.float32),
)


def kernel(node_emb, W, W_r, a, rel_emb, edge_index, edge_type):
    xt, scores, rel_s = _pre(node_emb, W, W_r, a, rel_emb)
    msg, att = _sc(xt, scores[:, 0], scores[:, 1], rel_s.reshape(-1),
                   edge_index[0], edge_index[1], edge_type)
    final = _post(msg, att, node_emb)
    return (final[:_NU], final[_NU:])
